# baseline (speedup 1.0000x reference)
import jax
import jax.numpy as jnp
from jax import lax
from jax.experimental import pallas as pl
from jax.experimental.pallas import tpu as pltpu
from jax.experimental.pallas import tpu_sc as plsc

SEQ = 512
B = 4096
TOT = 1048576
CLS_ID = 101
SEP_ID = 102
LIMIT = SEQ - 3
FLOOR_HALF = LIMIT // 2
CEIL_HALF = LIMIT - FLOOR_HALF

NC = 2
NS = 16
NW = NC * NS
RPW = B // NW
WIN = 520
PADF = 16
BUF = 1056
NSLOT = 4
BASEW = 264
EXTW = WIN - BASEW
G = 8
GW = G * SEQ
RBLK = 256


def _sc_body(tok_a, cu_a, tok_b, cu_b, out_w,
             cua_v, cub_v,
             ba0, ba1, ba2, ba3, bb0, bb1, bb2, bb3,
             w0, w1, semi, semo):
    bufa = (ba0, ba1, ba2, ba3)
    bufb = (bb0, bb1, bb2, bb3)
    wst = (w0, w1)

    cid = lax.axis_index("c")
    sid = lax.axis_index("s")
    wid = sid * NC + cid
    r0 = pl.multiple_of(wid * RPW, 8)

    pltpu.sync_copy(cu_a.at[pl.ds(r0, RPW + 8)], cua_v.at[pl.ds(0, RPW + 8)])
    pltpu.sync_copy(cu_b.at[pl.ds(r0, RPW + 8)], cub_v.at[pl.ds(0, RPW + 8)])

    def row_scalars(row):
        vca = cua_v[pl.ds(row, 16)]
        vcb = cub_v[pl.ds(row, 16)]
        sa0 = vca[0]
        sa1 = vca[1]
        sb0 = vcb[0]
        sb1 = vcb[1]
        la = sa1 - sa0
        lb = sb1 - sb0
        qa = jnp.minimum(la, CEIL_HALF + jnp.maximum(FLOOR_HALF - lb, 0))
        qb = jnp.minimum(lb, FLOOR_HALF + jnp.maximum(CEIL_HALF - la, 0))
        astart = pl.multiple_of(jnp.minimum(sa0 & ~7, TOT - WIN), 8)
        bstart = pl.multiple_of(jnp.minimum(sb0 & ~7, TOT - WIN), 8)
        pad_a = sa0 - astart
        pad_b = sb0 - bstart
        return qa, qb, astart, bstart, pad_a, pad_b

    def fetch(row, slot):
        qa, qb, astart, bstart, pad_a, pad_b = row_scalars(row)
        pltpu.async_copy(tok_a.at[pl.ds(astart, BASEW)],
                         bufa[slot].at[pl.ds(PADF, BASEW)], semi.at[slot, 0])
        pltpu.async_copy(tok_b.at[pl.ds(bstart, BASEW)],
                         bufb[slot].at[pl.ds(PADF, BASEW)], semi.at[slot, 1])

        @pl.when(pad_a + qa > BASEW)
        def _():
            pltpu.async_copy(tok_a.at[pl.ds(astart + BASEW, EXTW)],
                             bufa[slot].at[pl.ds(PADF + BASEW, EXTW)],
                             semi.at[slot, 0])

        @pl.when(pad_b + qb > BASEW)
        def _():
            pltpu.async_copy(tok_b.at[pl.ds(bstart + BASEW, EXTW)],
                             bufb[slot].at[pl.ds(PADF + BASEW, EXTW)],
                             semi.at[slot, 1])

    def wait_in(row, slot, qa, qb, pad_a, pad_b):
        pltpu.make_async_copy(tok_a.at[pl.ds(0, BASEW)],
                              bufa[slot].at[pl.ds(PADF, BASEW)],
                              semi.at[slot, 0]).wait()

        @pl.when(pad_a + qa > BASEW)
        def _():
            pltpu.make_async_copy(tok_a.at[pl.ds(0, EXTW)],
                                  bufa[slot].at[pl.ds(PADF + BASEW, EXTW)],
                                  semi.at[slot, 0]).wait()

        pltpu.make_async_copy(tok_b.at[pl.ds(0, BASEW)],
                              bufb[slot].at[pl.ds(PADF, BASEW)],
                              semi.at[slot, 1]).wait()

        @pl.when(pad_b + qb > BASEW)
        def _():
            pltpu.make_async_copy(tok_b.at[pl.ds(0, EXTW)],
                                  bufb[slot].at[pl.ds(PADF + BASEW, EXTW)],
                                  semi.at[slot, 1]).wait()

    def compute(row, slot, set_, k):
        qa, qb, astart, bstart, pad_a, pad_b = row_scalars(row)
        wait_in(row, slot, qa, qb, pad_a, pad_b)
        c1 = 1 + qa
        c2 = 2 + qa + qb
        wrow = wst[set_]
        ko = k * SEQ

        nb = c2 // 16 + 1

        @pl.loop(0, nb)
        def _(j):
            j16 = j * 16
            pos = lax.iota(jnp.int32, 16) + j16
            va = bufa[slot][pl.ds(pad_a + j16 + (PADF - 1), 16)]
            bb = jnp.maximum(pad_b + j16 + (PADF - 2) - qa, 0)
            vb = bufb[slot][pl.ds(bb, 16)]
            w = jnp.where(pos < c1, va,
                jnp.where(pos == c1, SEP_ID,
                jnp.where(pos < c2, vb,
                jnp.where(pos == c2, SEP_ID, 0))))
            w = jnp.where(pos == 0, CLS_ID, w)
            wrow[pl.ds(ko + j16, 16)] = w

        zeros = jnp.zeros((16,), jnp.int32)

        @pl.loop(nb, SEQ // 16)
        def _(j):
            wrow[pl.ds(ko + j * 16, 16)] = zeros

    def flush(base, set_):
        ro = pl.multiple_of((r0 + base) * SEQ, 8)
        pltpu.async_copy(wst[set_], out_w.at[pl.ds(ro, GW)], semo.at[set_])

    def wait_out(set_):
        pltpu.make_async_copy(wst[set_], out_w.at[pl.ds(0, GW)],
                              semo.at[set_]).wait()

    for s in range(NSLOT):
        fetch(s, s)

    @pl.loop(0, RPW, step=2 * G)
    def _(i):
        for set_ in range(2):
            base = i + set_ * G

            @pl.when(base >= 2 * G)
            def _():
                wait_out(set_)

            for k in range(G):
                row = base + k
                slot = (set_ * G + k) % NSLOT
                compute(row, slot, set_, k)
                nxt = row + NSLOT

                @pl.when(nxt < RPW)
                def _():
                    fetch(nxt, slot)

            flush(base, set_)

    wait_out(0)
    wait_out(1)


def _tc_body(la_ref, lb_ref, m_ref, t_ref):
    la = la_ref[...]
    lb = lb_ref[...]
    qa = jnp.minimum(la, CEIL_HALF + jnp.maximum(FLOOR_HALF - lb, 0))
    qb = jnp.minimum(lb, FLOOR_HALF + jnp.maximum(CEIL_HALF - la, 0))
    c1 = 1 + qa
    c2 = 2 + qa + qb
    pos = lax.broadcasted_iota(jnp.int32, (RBLK, SEQ), 1)
    m_ref[...] = jnp.where(pos <= c2, 1, 0)
    t_ref[...] = jnp.where((pos > c1) & (pos <= c2), 1, 0)


def kernel(tokens_a, cu_seqlens_a, tokens_b, cu_seqlens_b):
    cu_a32 = cu_seqlens_a.astype(jnp.int32)
    cu_b32 = cu_seqlens_b.astype(jnp.int32)
    cu_a = jnp.pad(cu_a32, (0, 7))
    cu_b = jnp.pad(cu_b32, (0, 7))
    mesh = plsc.VectorSubcoreMesh(core_axis_name="c", subcore_axis_name="s")
    out = jax.ShapeDtypeStruct((B * SEQ,), jnp.int32)
    sc = pl.kernel(
        _sc_body,
        out_type=out,
        mesh=mesh,
        scratch_types=(
            [pltpu.VMEM((RPW + 16,), jnp.int32)] * 2
            + [pltpu.VMEM((BUF,), jnp.int32)] * (2 * NSLOT)
            + [pltpu.VMEM((GW,), jnp.int32)] * 2
            + [pltpu.SemaphoreType.DMA((NSLOT, 2)),
               pltpu.SemaphoreType.DMA((2,))]
        ),
    )
    w = sc(tokens_a.astype(jnp.int32), cu_a, tokens_b.astype(jnp.int32), cu_b)

    la = (cu_a32[1:] - cu_a32[:-1]).reshape(B, 1)
    lb = (cu_b32[1:] - cu_b32[:-1]).reshape(B, 1)
    m, t = pl.pallas_call(
        _tc_body,
        out_shape=(jax.ShapeDtypeStruct((B, SEQ), jnp.int32),
                   jax.ShapeDtypeStruct((B, SEQ), jnp.int32)),
        grid=(B // RBLK,),
        in_specs=[pl.BlockSpec((RBLK, 1), lambda i: (i, 0)),
                  pl.BlockSpec((RBLK, 1), lambda i: (i, 0))],
        out_specs=(pl.BlockSpec((RBLK, SEQ), lambda i: (i, 0)),
                   pl.BlockSpec((RBLK, SEQ), lambda i: (i, 0))),
    )(la, lb)
    return (w.reshape(B, SEQ), m, t)


# speedup vs baseline: 1.0448x; 1.0448x over previous
import jax
import jax.numpy as jnp
from jax import lax
from jax.experimental import pallas as pl
from jax.experimental.pallas import tpu as pltpu
from jax.experimental.pallas import tpu_sc as plsc

SEQ = 512
B = 4096
TOT = 1048576
CLS_ID = 101
SEP_ID = 102
LIMIT = SEQ - 3
FLOOR_HALF = LIMIT // 2
CEIL_HALF = LIMIT - FLOOR_HALF

NC = 2
NS = 16
NW = NC * NS
RPW = B // NW
WIN = 520
PADF = 16
CHW = 8192
MAXCH = 7
CAPW = CHW * MAXCH
FB = PADF + CAPW
BUFW = FB + 16 + WIN + 552
G = 8
GW = G * SEQ
RBLK = 256


def _sc_body(tok_a, cu_a, tok_b, cu_b, out_w,
             cua_v, cub_v, ba, bb, w0, w1, semc, semo):
    wst = (w0, w1)

    cid = lax.axis_index("c")
    sid = lax.axis_index("s")
    wid = sid * NC + cid
    r0 = pl.multiple_of(wid * RPW, 8)

    pltpu.sync_copy(cu_a.at[pl.ds(r0, RPW + 8)], cua_v.at[pl.ds(0, RPW + 8)])
    pltpu.sync_copy(cu_b.at[pl.ds(r0, RPW + 8)], cub_v.at[pl.ds(0, RPW + 8)])

    def stream_setup(cuv, tok, buf, sidx):
        s0 = cuv[pl.ds(0, 16)][0]
        send = cuv[pl.ds(RPW, 16)][0]
        base = pl.multiple_of(s0 & ~7, 8)
        span = jnp.minimum(send + WIN, TOT) - base
        lcov = jnp.minimum(span, CAPW)
        nch = (lcov + (CHW - 1)) // CHW
        for t in range(MAXCH):
            @pl.when(t < nch)
            def _():
                src = pl.multiple_of(
                    jnp.minimum(base + t * CHW, TOT - CHW), 8)
                dst = pl.multiple_of(PADF + (src - base), 8)
                pltpu.async_copy(tok.at[pl.ds(src, CHW)],
                                 buf.at[pl.ds(dst, CHW)], semc.at[t, sidx])
        return base, lcov, nch

    base_a, lcov_a, nch_a = stream_setup(cua_v, tok_a, ba, 0)
    base_b, lcov_b, nch_b = stream_setup(cub_v, tok_b, bb, 1)

    def wait_chunks(waited, needed, tok, buf, sidx):
        for t in range(MAXCH):
            @pl.when((t >= waited) & (t < needed))
            def _():
                pltpu.make_async_copy(tok.at[pl.ds(0, CHW)],
                                      buf.at[pl.ds(PADF, CHW)],
                                      semc.at[t, sidx]).wait()

    def compute(row, set_, k):
        vca = cua_v[pl.ds(row, 16)]
        vcb = cub_v[pl.ds(row, 16)]
        sa0 = vca[0]
        sa1 = vca[1]
        sb0 = vcb[0]
        sb1 = vcb[1]
        la = sa1 - sa0
        lb = sb1 - sb0
        qa = jnp.minimum(la, CEIL_HALF + jnp.maximum(FLOOR_HALF - lb, 0))
        qb = jnp.minimum(lb, FLOOR_HALF + jnp.maximum(CEIL_HALF - la, 0))
        c1 = 1 + qa
        c2 = 2 + qa + qb

        cov_a = (sa0 - base_a) + WIN <= lcov_a
        cov_b = (sb0 - base_b) + WIN <= lcov_b

        @pl.when(jnp.logical_not(cov_a))
        def _():
            astart = pl.multiple_of(jnp.minimum(sa0 & ~7, TOT - WIN), 8)
            pltpu.sync_copy(tok_a.at[pl.ds(astart, WIN)],
                            ba.at[pl.ds(FB + PADF, WIN)])

        @pl.when(jnp.logical_not(cov_b))
        def _():
            bstart = pl.multiple_of(jnp.minimum(sb0 & ~7, TOT - WIN), 8)
            pltpu.sync_copy(tok_b.at[pl.ds(bstart, WIN)],
                            bb.at[pl.ds(FB + PADF, WIN)])

        off_a = jnp.where(cov_a, sa0 - base_a,
                          FB + (sa0 - jnp.minimum(sa0 & ~7, TOT - WIN)))
        off_b = jnp.where(cov_b, sb0 - base_b,
                          FB + (sb0 - jnp.minimum(sb0 & ~7, TOT - WIN)))

        wrow = wst[set_]
        ko = k * SEQ
        nb = c2 // 16 + 1

        @pl.loop(0, nb)
        def _(j):
            j16 = j * 16
            pos = lax.iota(jnp.int32, 16) + j16
            va = ba[pl.ds(off_a + j16 + (PADF - 1), 16)]
            bbase = jnp.maximum(off_b + j16 + (PADF - 2) - qa, 0)
            vb = bb[pl.ds(bbase, 16)]
            w = jnp.where(pos < c1, va,
                jnp.where(pos == c1, SEP_ID,
                jnp.where(pos < c2, vb,
                jnp.where(pos == c2, SEP_ID, 0))))
            w = jnp.where(pos == 0, CLS_ID, w)
            wrow[pl.ds(ko + j16, 16)] = w

        zeros = jnp.zeros((16,), jnp.int32)

        @pl.loop(nb, SEQ // 16)
        def _(j):
            wrow[pl.ds(ko + j * 16, 16)] = zeros

    def flush(base, set_):
        ro = pl.multiple_of((r0 + base) * SEQ, 8)
        pltpu.async_copy(wst[set_], out_w.at[pl.ds(ro, GW)], semo.at[set_])

    def wait_out(set_):
        pltpu.make_async_copy(wst[set_], out_w.at[pl.ds(0, GW)],
                              semo.at[set_]).wait()

    @pl.loop(0, RPW, step=2 * G,
             init_carry=(jnp.int32(0), jnp.int32(0)))
    def final_waited(i, carry):
        wa, wb = carry
        sa_last = cua_v[pl.ds(i + 2 * G - 1, 16)][0]
        sb_last = cub_v[pl.ds(i + 2 * G - 1, 16)][0]
        need_a = jnp.minimum((sa_last - base_a + WIN + (CHW - 1)) // CHW, nch_a)
        need_b = jnp.minimum((sb_last - base_b + WIN + (CHW - 1)) // CHW, nch_b)
        wait_chunks(wa, need_a, tok_a, ba, 0)
        wait_chunks(wb, need_b, tok_b, bb, 1)

        for set_ in range(2):
            base = i + set_ * G

            @pl.when(base >= 2 * G)
            def _():
                wait_out(set_)

            for k in range(G):
                compute(base + k, set_, k)

            flush(base, set_)

        return (jnp.maximum(wa, need_a), jnp.maximum(wb, need_b))

    wait_chunks(final_waited[0], nch_a, tok_a, ba, 0)
    wait_chunks(final_waited[1], nch_b, tok_b, bb, 1)
    wait_out(0)
    wait_out(1)


def _tc_body(la_ref, lb_ref, m_ref, t_ref):
    la = la_ref[...]
    lb = lb_ref[...]
    qa = jnp.minimum(la, CEIL_HALF + jnp.maximum(FLOOR_HALF - lb, 0))
    qb = jnp.minimum(lb, FLOOR_HALF + jnp.maximum(CEIL_HALF - la, 0))
    c1 = 1 + qa
    c2 = 2 + qa + qb
    pos = lax.broadcasted_iota(jnp.int32, (RBLK, SEQ), 1)
    m_ref[...] = jnp.where(pos <= c2, 1, 0)
    t_ref[...] = jnp.where((pos > c1) & (pos <= c2), 1, 0)


def kernel(tokens_a, cu_seqlens_a, tokens_b, cu_seqlens_b):
    cu_a32 = cu_seqlens_a.astype(jnp.int32)
    cu_b32 = cu_seqlens_b.astype(jnp.int32)
    cu_a = jnp.pad(cu_a32, (0, 7))
    cu_b = jnp.pad(cu_b32, (0, 7))
    mesh = plsc.VectorSubcoreMesh(core_axis_name="c", subcore_axis_name="s")
    out = jax.ShapeDtypeStruct((B * SEQ,), jnp.int32)
    sc = pl.kernel(
        _sc_body,
        out_type=out,
        mesh=mesh,
        scratch_types=(
            [pltpu.VMEM((RPW + 16,), jnp.int32)] * 2
            + [pltpu.VMEM((BUFW,), jnp.int32)] * 2
            + [pltpu.VMEM((GW,), jnp.int32)] * 2
            + [pltpu.SemaphoreType.DMA((MAXCH, 2)),
               pltpu.SemaphoreType.DMA((2,))]
        ),
    )
    w = sc(tokens_a.astype(jnp.int32), cu_a, tokens_b.astype(jnp.int32), cu_b)

    la = (cu_a32[1:] - cu_a32[:-1]).reshape(B, 1)
    lb = (cu_b32[1:] - cu_b32[:-1]).reshape(B, 1)
    m, t = pl.pallas_call(
        _tc_body,
        out_shape=(jax.ShapeDtypeStruct((B, SEQ), jnp.int32),
                   jax.ShapeDtypeStruct((B, SEQ), jnp.int32)),
        grid=(B // RBLK,),
        in_specs=[pl.BlockSpec((RBLK, 1), lambda i: (i, 0)),
                  pl.BlockSpec((RBLK, 1), lambda i: (i, 0))],
        out_specs=(pl.BlockSpec((RBLK, SEQ), lambda i: (i, 0)),
                   pl.BlockSpec((RBLK, SEQ), lambda i: (i, 0))),
    )(la, lb)
    return (w.reshape(B, SEQ), m, t)


# speedup vs baseline: 1.0697x; 1.0239x over previous
import jax
import jax.numpy as jnp
from jax import lax
from jax.experimental import pallas as pl
from jax.experimental.pallas import tpu as pltpu
from jax.experimental.pallas import tpu_sc as plsc

SEQ = 512
B = 4096
TOT = 1048576
CLS_ID = 101
SEP_ID = 102
LIMIT = SEQ - 3
FLOOR_HALF = LIMIT // 2
CEIL_HALF = LIMIT - FLOOR_HALF

NC = 2
NS = 16
NW = NC * NS
RPW = B // NW
WIN = 520
PADF = 16
CHW = 8192
MAXCH = 7
CAPW = CHW * MAXCH
FB = PADF + CAPW
BUFW = FB + 16 + WIN + 552
G = 8
GW = G * SEQ
RBLK = 256


def _sc_body(tok_a, cu_a, tok_b, cu_b, out_w,
             cua_v, cub_v, ba, bb, w0, w1, semc, semo):
    wst = (w0, w1)

    cid = lax.axis_index("c")
    sid = lax.axis_index("s")
    wid = sid * NC + cid
    r0 = pl.multiple_of(wid * RPW, 8)

    pltpu.sync_copy(cu_a.at[pl.ds(r0, RPW + 8)], cua_v.at[pl.ds(0, RPW + 8)])
    pltpu.sync_copy(cu_b.at[pl.ds(r0, RPW + 8)], cub_v.at[pl.ds(0, RPW + 8)])

    def stream_setup(cuv, tok, buf, sidx):
        s0 = cuv[pl.ds(0, 16)][0]
        send = cuv[pl.ds(RPW, 16)][0]
        base = pl.multiple_of(s0 & ~7, 8)
        span = jnp.minimum(send + WIN, TOT) - base
        lcov = jnp.minimum(span, CAPW)
        nch = (lcov + (CHW - 1)) // CHW
        for t in range(MAXCH):
            @pl.when(t < nch)
            def _():
                src = pl.multiple_of(
                    jnp.minimum(base + t * CHW, TOT - CHW), 8)
                dst = pl.multiple_of(PADF + (src - base), 8)
                pltpu.async_copy(tok.at[pl.ds(src, CHW)],
                                 buf.at[pl.ds(dst, CHW)], semc.at[t, sidx])
        return base, lcov, nch

    base_a, lcov_a, nch_a = stream_setup(cua_v, tok_a, ba, 0)
    base_b, lcov_b, nch_b = stream_setup(cub_v, tok_b, bb, 1)

    def wait_chunks(waited, needed, tok, buf, sidx):
        for t in range(MAXCH):
            @pl.when((t >= waited) & (t < needed))
            def _():
                pltpu.make_async_copy(tok.at[pl.ds(0, CHW)],
                                      buf.at[pl.ds(PADF, CHW)],
                                      semc.at[t, sidx]).wait()

    def compute(row, set_, k):
        vca = cua_v[pl.ds(row, 16)]
        vcb = cub_v[pl.ds(row, 16)]
        sa0 = vca[0]
        sa1 = vca[1]
        sb0 = vcb[0]
        sb1 = vcb[1]
        la = sa1 - sa0
        lb = sb1 - sb0
        qa = jnp.minimum(la, CEIL_HALF + jnp.maximum(FLOOR_HALF - lb, 0))
        qb = jnp.minimum(lb, FLOOR_HALF + jnp.maximum(CEIL_HALF - la, 0))
        c1 = 1 + qa
        c2 = 2 + qa + qb

        cov_a = (sa0 - base_a) + WIN <= lcov_a
        cov_b = (sb0 - base_b) + WIN <= lcov_b

        @pl.when(jnp.logical_not(cov_a))
        def _():
            astart = pl.multiple_of(jnp.minimum(sa0 & ~7, TOT - WIN), 8)
            pltpu.sync_copy(tok_a.at[pl.ds(astart, WIN)],
                            ba.at[pl.ds(FB + PADF, WIN)])

        @pl.when(jnp.logical_not(cov_b))
        def _():
            bstart = pl.multiple_of(jnp.minimum(sb0 & ~7, TOT - WIN), 8)
            pltpu.sync_copy(tok_b.at[pl.ds(bstart, WIN)],
                            bb.at[pl.ds(FB + PADF, WIN)])

        off_a = jnp.where(cov_a, sa0 - base_a,
                          FB + (sa0 - jnp.minimum(sa0 & ~7, TOT - WIN)))
        off_b = jnp.where(cov_b, sb0 - base_b,
                          FB + (sb0 - jnp.minimum(sb0 & ~7, TOT - WIN)))

        wrow = wst[set_]
        ko = k * SEQ
        nb = c2 // 16 + 1

        def wblock(j16, pos):
            va = ba[pl.ds(off_a + j16 + (PADF - 1), 16)]
            bbase = jnp.maximum(off_b + j16 + (PADF - 2) - qa, 0)
            vb = bb[pl.ds(bbase, 16)]
            sep = (pos == c1) | (pos == c2)
            return jnp.where(pos < c1, va,
                   jnp.where(sep, SEP_ID,
                   jnp.where(pos < c2, vb, 0)))

        pos0 = lax.iota(jnp.int32, 16)
        w0v = jnp.where(pos0 == 0, CLS_ID, wblock(0, pos0))
        wrow[pl.ds(ko, 16)] = w0v

        @pl.loop(1, nb)
        def _(j):
            j16 = j * 16
            pos = lax.iota(jnp.int32, 16) + j16
            wrow[pl.ds(ko + j16, 16)] = wblock(j16, pos)

        zeros = jnp.zeros((16,), jnp.int32)

        @pl.loop(nb, SEQ // 16)
        def _(j):
            wrow[pl.ds(ko + j * 16, 16)] = zeros

    def flush(base, set_):
        ro = pl.multiple_of((r0 + base) * SEQ, 8)
        pltpu.async_copy(wst[set_], out_w.at[pl.ds(ro, GW)], semo.at[set_])

    def wait_out(set_):
        pltpu.make_async_copy(wst[set_], out_w.at[pl.ds(0, GW)],
                              semo.at[set_]).wait()

    @pl.loop(0, RPW, step=2 * G,
             init_carry=(jnp.int32(0), jnp.int32(0)))
    def final_waited(i, carry):
        wa, wb = carry
        sa_last = cua_v[pl.ds(i + 2 * G - 1, 16)][0]
        sb_last = cub_v[pl.ds(i + 2 * G - 1, 16)][0]
        need_a = jnp.minimum((sa_last - base_a + WIN + (CHW - 1)) // CHW, nch_a)
        need_b = jnp.minimum((sb_last - base_b + WIN + (CHW - 1)) // CHW, nch_b)
        wait_chunks(wa, need_a, tok_a, ba, 0)
        wait_chunks(wb, need_b, tok_b, bb, 1)

        for set_ in range(2):
            base = i + set_ * G

            @pl.when(base >= 2 * G)
            def _():
                wait_out(set_)

            for k in range(G):
                compute(base + k, set_, k)

            flush(base, set_)

        return (jnp.maximum(wa, need_a), jnp.maximum(wb, need_b))

    wait_chunks(final_waited[0], nch_a, tok_a, ba, 0)
    wait_chunks(final_waited[1], nch_b, tok_b, bb, 1)
    wait_out(0)
    wait_out(1)


def _tc_body(la_ref, lb_ref, m_ref, t_ref):
    la = la_ref[...]
    lb = lb_ref[...]
    qa = jnp.minimum(la, CEIL_HALF + jnp.maximum(FLOOR_HALF - lb, 0))
    qb = jnp.minimum(lb, FLOOR_HALF + jnp.maximum(CEIL_HALF - la, 0))
    c1 = 1 + qa
    c2 = 2 + qa + qb
    pos = lax.broadcasted_iota(jnp.int32, (RBLK, SEQ), 1)
    m_ref[...] = jnp.where(pos <= c2, 1, 0)
    t_ref[...] = jnp.where((pos > c1) & (pos <= c2), 1, 0)


def kernel(tokens_a, cu_seqlens_a, tokens_b, cu_seqlens_b):
    cu_a32 = cu_seqlens_a.astype(jnp.int32)
    cu_b32 = cu_seqlens_b.astype(jnp.int32)
    cu_a = jnp.pad(cu_a32, (0, 7))
    cu_b = jnp.pad(cu_b32, (0, 7))
    mesh = plsc.VectorSubcoreMesh(core_axis_name="c", subcore_axis_name="s")
    out = jax.ShapeDtypeStruct((B * SEQ,), jnp.int32)
    sc = pl.kernel(
        _sc_body,
        out_type=out,
        mesh=mesh,
        scratch_types=(
            [pltpu.VMEM((RPW + 16,), jnp.int32)] * 2
            + [pltpu.VMEM((BUFW,), jnp.int32)] * 2
            + [pltpu.VMEM((GW,), jnp.int32)] * 2
            + [pltpu.SemaphoreType.DMA((MAXCH, 2)),
               pltpu.SemaphoreType.DMA((2,))]
        ),
    )
    w = sc(tokens_a.astype(jnp.int32), cu_a, tokens_b.astype(jnp.int32), cu_b)

    la = (cu_a32[1:] - cu_a32[:-1]).reshape(B, 1)
    lb = (cu_b32[1:] - cu_b32[:-1]).reshape(B, 1)
    m, t = pl.pallas_call(
        _tc_body,
        out_shape=(jax.ShapeDtypeStruct((B, SEQ), jnp.int32),
                   jax.ShapeDtypeStruct((B, SEQ), jnp.int32)),
        grid=(B // RBLK,),
        in_specs=[pl.BlockSpec((RBLK, 1), lambda i: (i, 0)),
                  pl.BlockSpec((RBLK, 1), lambda i: (i, 0))],
        out_specs=(pl.BlockSpec((RBLK, SEQ), lambda i: (i, 0)),
                   pl.BlockSpec((RBLK, SEQ), lambda i: (i, 0))),
    )(la, lb)
    return (w.reshape(B, SEQ), m, t)
